# baseline (speedup 1.0000x reference)
import functools

import jax
import jax.numpy as jnp
from jax import lax
from jax.experimental import pallas as pl
from jax.experimental.pallas import tpu as pltpu
from jax.experimental.pallas import tpu_sc as plsc

_NUM_EMB = 8192
_EMB_DIM = 256
_N_ROWS = 16384

_BI = 256
_NI = _N_ROWS // _BI

_NW = 32
_ROWS_PER_WORKER = _N_ROWS // _NW
_GATHER_CHUNK = 128
_N_CHUNKS = _ROWS_PER_WORKER // _GATHER_CHUNK


def _vq_body(x_ref, w_ref, cj_ref, g_ref, loss_ref, idx_ref, oh_ref):
    i = pl.program_id(0)

    @pl.when(i == 0)
    def _():
        loss_ref[0, 0] = 0.0

    d = lax.dot_general(
        x_ref[...], w_ref[...],
        dimension_numbers=(((1,), (1,)), ((), ())),
        preferred_element_type=jnp.float32,
    )
    m = jnp.max(d, axis=1, keepdims=True)
    eq = d == m
    ohf = eq.astype(jnp.float32)
    sums = lax.dot_general(
        eq.astype(jnp.bfloat16), cj_ref[...],
        dimension_numbers=(((1,), (1,)), ((), ())),
        preferred_element_type=jnp.float32,
    )
    cnt_max = jnp.max(sums[:, 2:3])

    @pl.when(cnt_max < 1.5)
    def _():
        idx_ref[...] = (64.0 * sums[:, 0:1] + sums[:, 1:2]).astype(jnp.int32)
        oh_ref[...] = ohf

    @pl.when(cnt_max >= 1.5)
    def _():
        cols = lax.broadcasted_iota(jnp.int32, d.shape, 1)
        la = jnp.min(jnp.where(eq, cols, _NUM_EMB), axis=1, keepdims=True)
        idx_ref[...] = la
        oh_ref[...] = jnp.where(cols == la, 1.0, 0.0)

    loss_ref[0, 0] += jnp.sum(x_ref[...] * g_ref[...])

    @pl.when(i == _NI - 1)
    def _():
        loss_ref[0, 0] = 1.0 - loss_ref[0, 0] / float(_N_ROWS)


_vq_call = pl.pallas_call(
    _vq_body,
    grid=(_NI,),
    in_specs=[
        pl.BlockSpec((_BI, _EMB_DIM), lambda i: (i, 0)),
        pl.BlockSpec((_NUM_EMB, _EMB_DIM), lambda i: (0, 0)),
        pl.BlockSpec((3, _NUM_EMB), lambda i: (0, 0)),
        pl.BlockSpec((_BI, _EMB_DIM), lambda i: (i, 0)),
    ],
    out_specs=[
        pl.BlockSpec((1, 1), lambda i: (0, 0), memory_space=pltpu.SMEM),
        pl.BlockSpec((_BI, 1), lambda i: (i, 0)),
        pl.BlockSpec((_BI, _NUM_EMB), lambda i: (i, 0)),
    ],
    out_shape=[
        jax.ShapeDtypeStruct((1, 1), jnp.float32),
        jax.ShapeDtypeStruct((_N_ROWS, 1), jnp.int32),
        jax.ShapeDtypeStruct((_N_ROWS, _NUM_EMB), jnp.float32),
    ],
)


@functools.cache
def _make_sc_gather():
    @functools.partial(
        pl.kernel,
        mesh=plsc.VectorSubcoreMesh(core_axis_name="c", subcore_axis_name="s"),
        out_type=jax.ShapeDtypeStruct((_N_ROWS, _EMB_DIM), jnp.float32),
        scratch_types=[
            pltpu.VMEM((_GATHER_CHUNK,), jnp.int32),
            pltpu.VMEM((_GATHER_CHUNK, _EMB_DIM), jnp.float32),
            pltpu.SemaphoreType.DMA,
        ],
    )
    def _sc_gather(table_hbm, idx_hbm, out_hbm, idx_v, rows_v, sem):
        wid = lax.axis_index("s") * 2 + lax.axis_index("c")
        base = wid * _ROWS_PER_WORKER
        for c in range(_N_CHUNKS):
            off = base + c * _GATHER_CHUNK
            pltpu.sync_copy(idx_hbm.at[pl.ds(off, _GATHER_CHUNK)], idx_v)
            pltpu.async_copy(table_hbm.at[idx_v], rows_v, sem).wait()
            pltpu.sync_copy(rows_v, out_hbm.at[pl.ds(off, _GATHER_CHUNK)])

    return _sc_gather


def _l2norm(t):
    n = jnp.linalg.norm(t, axis=1, keepdims=True)
    return t / jnp.maximum(n, 1e-12)


def kernel(inputs, labels, W):
    flat = inputs.reshape(-1, _EMB_DIM)
    xn = _l2norm(flat)
    wn = _l2norm(W)
    labels_i32 = labels.astype(jnp.int32)
    cols = jnp.arange(_NUM_EMB, dtype=jnp.int32)
    cj = jnp.stack([
        (cols // 64).astype(jnp.bfloat16),
        (cols % 64).astype(jnp.bfloat16),
        jnp.ones((_NUM_EMB,), jnp.bfloat16),
    ])

    sc_gather = _make_sc_gather()
    lab_rows = sc_gather(wn, labels_i32)

    loss2d, idx2d, encodings = _vq_call(xn, wn, cj, lab_rows)

    quantized = sc_gather(W, idx2d.reshape(_N_ROWS))

    return (
        loss2d.reshape(()),
        quantized.reshape(inputs.shape),
        jnp.array(1),
        encodings,
        idx2d,
    )


# speedup vs baseline: 1.0840x; 1.0840x over previous
import functools

import jax
import jax.numpy as jnp
from jax import lax
from jax.experimental import pallas as pl
from jax.experimental.pallas import tpu as pltpu
from jax.experimental.pallas import tpu_sc as plsc

_NUM_EMB = 8192
_EMB_DIM = 256
_N_ROWS = 16384

_BI = 256
_NI = _N_ROWS // _BI

_NW = 32
_ROWS_PER_WORKER = _N_ROWS // _NW
_GATHER_CHUNK = 128
_N_CHUNKS = _ROWS_PER_WORKER // _GATHER_CHUNK


def _vq_body(x_ref, w_ref, cj_ref, g_ref, loss_ref, idx_ref, oh_ref):
    i = pl.program_id(0)

    @pl.when(i == 0)
    def _():
        loss_ref[0, 0] = 0.0

    d = lax.dot_general(
        x_ref[...], w_ref[...],
        dimension_numbers=(((1,), (1,)), ((), ())),
        preferred_element_type=jnp.float32,
    )
    m = jnp.max(d, axis=1, keepdims=True)
    eq = d == m
    ohf = eq.astype(jnp.float32)
    sums = lax.dot_general(
        eq.astype(jnp.bfloat16), cj_ref[...],
        dimension_numbers=(((1,), (1,)), ((), ())),
        preferred_element_type=jnp.float32,
    )
    idx_ref[...] = (64.0 * sums[:, 0:1] + sums[:, 1:2]).astype(jnp.int32)
    oh_ref[...] = ohf

    @pl.when(jnp.max(sums[:, 2:3]) >= 1.5)
    def _():
        cols = lax.broadcasted_iota(jnp.int32, d.shape, 1)
        la = jnp.min(jnp.where(eq, cols, _NUM_EMB), axis=1, keepdims=True)
        idx_ref[...] = la
        oh_ref[...] = jnp.where(cols == la, 1.0, 0.0)

    loss_ref[0, 0] += jnp.sum(x_ref[...] * g_ref[...])

    @pl.when(i == _NI - 1)
    def _():
        loss_ref[0, 0] = 1.0 - loss_ref[0, 0] / float(_N_ROWS)


_vq_call = pl.pallas_call(
    _vq_body,
    grid=(_NI,),
    in_specs=[
        pl.BlockSpec((_BI, _EMB_DIM), lambda i: (i, 0)),
        pl.BlockSpec((_NUM_EMB, _EMB_DIM), lambda i: (0, 0)),
        pl.BlockSpec((3, _NUM_EMB), lambda i: (0, 0)),
        pl.BlockSpec((_BI, _EMB_DIM), lambda i: (i, 0)),
    ],
    out_specs=[
        pl.BlockSpec((1, 1), lambda i: (0, 0), memory_space=pltpu.SMEM),
        pl.BlockSpec((_BI, 1), lambda i: (i, 0)),
        pl.BlockSpec((_BI, _NUM_EMB), lambda i: (i, 0)),
    ],
    out_shape=[
        jax.ShapeDtypeStruct((1, 1), jnp.float32),
        jax.ShapeDtypeStruct((_N_ROWS, 1), jnp.int32),
        jax.ShapeDtypeStruct((_N_ROWS, _NUM_EMB), jnp.float32),
    ],
)


@functools.cache
def _make_sc_gather():
    @functools.partial(
        pl.kernel,
        mesh=plsc.VectorSubcoreMesh(core_axis_name="c", subcore_axis_name="s"),
        out_type=jax.ShapeDtypeStruct((_N_ROWS, _EMB_DIM), jnp.float32),
        scratch_types=[
            pltpu.VMEM((_GATHER_CHUNK,), jnp.int32),
            pltpu.VMEM((_GATHER_CHUNK, _EMB_DIM), jnp.float32),
            pltpu.SemaphoreType.DMA,
        ],
    )
    def _sc_gather(table_hbm, idx_hbm, out_hbm, idx_v, rows_v, sem):
        wid = lax.axis_index("s") * 2 + lax.axis_index("c")
        base = wid * _ROWS_PER_WORKER
        for c in range(_N_CHUNKS):
            off = base + c * _GATHER_CHUNK
            pltpu.sync_copy(idx_hbm.at[pl.ds(off, _GATHER_CHUNK)], idx_v)
            pltpu.async_copy(table_hbm.at[idx_v], rows_v, sem).wait()
            pltpu.sync_copy(rows_v, out_hbm.at[pl.ds(off, _GATHER_CHUNK)])

    return _sc_gather


def _l2norm(t):
    n = jnp.linalg.norm(t, axis=1, keepdims=True)
    return t / jnp.maximum(n, 1e-12)


def kernel(inputs, labels, W):
    flat = inputs.reshape(-1, _EMB_DIM)
    xn = _l2norm(flat)
    wn = _l2norm(W)
    labels_i32 = labels.astype(jnp.int32)
    cols = jnp.arange(_NUM_EMB, dtype=jnp.int32)
    cj = jnp.stack([
        (cols // 64).astype(jnp.bfloat16),
        (cols % 64).astype(jnp.bfloat16),
        jnp.ones((_NUM_EMB,), jnp.bfloat16),
    ])

    sc_gather = _make_sc_gather()
    lab_rows = sc_gather(wn, labels_i32)

    loss2d, idx2d, encodings = _vq_call(xn, wn, cj, lab_rows)

    quantized = sc_gather(W, idx2d.reshape(_N_ROWS))

    return (
        loss2d.reshape(()),
        quantized.reshape(inputs.shape),
        jnp.array(1),
        encodings,
        idx2d,
    )


# speedup vs baseline: 1.5873x; 1.4643x over previous
import functools

import jax
import jax.numpy as jnp
from jax import lax
from jax.experimental import pallas as pl
from jax.experimental.pallas import tpu as pltpu
from jax.experimental.pallas import tpu_sc as plsc

_NUM_EMB = 8192
_EMB_DIM = 256
_N_ROWS = 16384

_BI = 256
_NI = _N_ROWS // _BI

_NW = 32
_ROWS_PER_WORKER = _N_ROWS // _NW
_GATHER_CHUNK = 128
_N_CHUNKS = _ROWS_PER_WORKER // _GATHER_CHUNK


def _vq_body(x_ref, w_ref, g_ref, loss_ref, idx_ref, oh_ref):
    i = pl.program_id(0)

    @pl.when(i == 0)
    def _():
        loss_ref[0, 0] = 0.0

    d = lax.dot_general(
        x_ref[...], w_ref[...],
        dimension_numbers=(((1,), (1,)), ((), ())),
        preferred_element_type=jnp.float32,
    )
    m = jnp.max(d, axis=1, keepdims=True)
    eq = d == m
    oh_ref[...] = eq.astype(jnp.float32)
    cols = lax.broadcasted_iota(jnp.int32, d.shape, 1)
    idx_ref[...] = jnp.min(jnp.where(eq, cols, _NUM_EMB), axis=1,
                           keepdims=True)

    loss_ref[0, 0] += jnp.sum(x_ref[...] * g_ref[...])

    @pl.when(i == _NI - 1)
    def _():
        loss_ref[0, 0] = 1.0 - loss_ref[0, 0] / float(_N_ROWS)


_vq_call = pl.pallas_call(
    _vq_body,
    grid=(_NI,),
    in_specs=[
        pl.BlockSpec((_BI, _EMB_DIM), lambda i: (i, 0)),
        pl.BlockSpec((_NUM_EMB, _EMB_DIM), lambda i: (0, 0)),
        pl.BlockSpec((_BI, _EMB_DIM), lambda i: (i, 0)),
    ],
    out_specs=[
        pl.BlockSpec((1, 1), lambda i: (0, 0), memory_space=pltpu.SMEM),
        pl.BlockSpec((_BI, 1), lambda i: (i, 0)),
        pl.BlockSpec((_BI, _NUM_EMB), lambda i: (i, 0)),
    ],
    out_shape=[
        jax.ShapeDtypeStruct((1, 1), jnp.float32),
        jax.ShapeDtypeStruct((_N_ROWS, 1), jnp.int32),
        jax.ShapeDtypeStruct((_N_ROWS, _NUM_EMB), jnp.float32),
    ],
)


@functools.cache
def _make_sc_gather():
    @functools.partial(
        pl.kernel,
        mesh=plsc.VectorSubcoreMesh(core_axis_name="c", subcore_axis_name="s"),
        out_type=jax.ShapeDtypeStruct((_N_ROWS, _EMB_DIM), jnp.float32),
        scratch_types=[
            pltpu.VMEM((_GATHER_CHUNK,), jnp.int32),
            pltpu.VMEM((_GATHER_CHUNK, _EMB_DIM), jnp.float32),
            pltpu.SemaphoreType.DMA,
        ],
    )
    def _sc_gather(table_hbm, idx_hbm, out_hbm, idx_v, rows_v, sem):
        wid = lax.axis_index("s") * 2 + lax.axis_index("c")
        base = wid * _ROWS_PER_WORKER
        for c in range(_N_CHUNKS):
            off = base + c * _GATHER_CHUNK
            pltpu.sync_copy(idx_hbm.at[pl.ds(off, _GATHER_CHUNK)], idx_v)
            pltpu.async_copy(table_hbm.at[idx_v], rows_v, sem).wait()
            pltpu.sync_copy(rows_v, out_hbm.at[pl.ds(off, _GATHER_CHUNK)])

    return _sc_gather


def _l2norm(t):
    n = jnp.linalg.norm(t, axis=1, keepdims=True)
    return t / jnp.maximum(n, 1e-12)


def kernel(inputs, labels, W):
    flat = inputs.reshape(-1, _EMB_DIM)
    xn = _l2norm(flat)
    wn = _l2norm(W)
    labels_i32 = labels.astype(jnp.int32)

    sc_gather = _make_sc_gather()
    lab_rows = sc_gather(wn, labels_i32)

    loss2d, idx2d, encodings = _vq_call(xn, wn, lab_rows)

    quantized = sc_gather(W, idx2d.reshape(_N_ROWS))

    return (
        loss2d.reshape(()),
        quantized.reshape(inputs.shape),
        jnp.array(1),
        encodings,
        idx2d,
    )


# speedup vs baseline: 1.6016x; 1.0090x over previous
import functools

import jax
import jax.numpy as jnp
from jax import lax
from jax.experimental import pallas as pl
from jax.experimental.pallas import tpu as pltpu
from jax.experimental.pallas import tpu_sc as plsc

_NUM_EMB = 8192
_EMB_DIM = 256
_N_ROWS = 16384

_BI = 256
_NI = _N_ROWS // _BI

_NW = 32
_ROWS_PER_WORKER = _N_ROWS // _NW
_GATHER_CHUNK = 128
_N_CHUNKS = _ROWS_PER_WORKER // _GATHER_CHUNK


def _vq_body(x_ref, w_ref, g_ref, loss_ref, idx_ref, oh_ref):
    i = pl.program_id(0)

    @pl.when(i == 0)
    def _():
        loss_ref[0, 0] = 0.0

    d = lax.dot_general(
        x_ref[...], w_ref[...],
        dimension_numbers=(((1,), (1,)), ((), ())),
        preferred_element_type=jnp.float32,
    )
    m = jnp.max(d, axis=1, keepdims=True)
    eq = d == m
    cols = lax.broadcasted_iota(jnp.int32, d.shape, 1)
    la = jnp.min(jnp.where(eq, cols, _NUM_EMB), axis=1, keepdims=True)
    idx_ref[...] = la
    oh_ref[...] = (cols == la).astype(jnp.float32)

    loss_ref[0, 0] += jnp.sum(x_ref[...] * g_ref[...])

    @pl.when(i == _NI - 1)
    def _():
        loss_ref[0, 0] = 1.0 - loss_ref[0, 0] / float(_N_ROWS)


_vq_call = pl.pallas_call(
    _vq_body,
    grid=(_NI,),
    in_specs=[
        pl.BlockSpec((_BI, _EMB_DIM), lambda i: (i, 0)),
        pl.BlockSpec((_NUM_EMB, _EMB_DIM), lambda i: (0, 0)),
        pl.BlockSpec((_BI, _EMB_DIM), lambda i: (i, 0)),
    ],
    out_specs=[
        pl.BlockSpec((1, 1), lambda i: (0, 0), memory_space=pltpu.SMEM),
        pl.BlockSpec((_BI, 1), lambda i: (i, 0)),
        pl.BlockSpec((_BI, _NUM_EMB), lambda i: (i, 0)),
    ],
    out_shape=[
        jax.ShapeDtypeStruct((1, 1), jnp.float32),
        jax.ShapeDtypeStruct((_N_ROWS, 1), jnp.int32),
        jax.ShapeDtypeStruct((_N_ROWS, _NUM_EMB), jnp.float32),
    ],
)


@functools.cache
def _make_sc_gather():
    @functools.partial(
        pl.kernel,
        mesh=plsc.VectorSubcoreMesh(core_axis_name="c", subcore_axis_name="s"),
        out_type=jax.ShapeDtypeStruct((_N_ROWS, _EMB_DIM), jnp.float32),
        scratch_types=[
            pltpu.VMEM((_GATHER_CHUNK,), jnp.int32),
            pltpu.VMEM((_GATHER_CHUNK, _EMB_DIM), jnp.float32),
            pltpu.SemaphoreType.DMA,
        ],
    )
    def _sc_gather(table_hbm, idx_hbm, out_hbm, idx_v, rows_v, sem):
        wid = lax.axis_index("s") * 2 + lax.axis_index("c")
        base = wid * _ROWS_PER_WORKER
        for c in range(_N_CHUNKS):
            off = base + c * _GATHER_CHUNK
            pltpu.sync_copy(idx_hbm.at[pl.ds(off, _GATHER_CHUNK)], idx_v)
            pltpu.async_copy(table_hbm.at[idx_v], rows_v, sem).wait()
            pltpu.sync_copy(rows_v, out_hbm.at[pl.ds(off, _GATHER_CHUNK)])

    return _sc_gather


def _l2norm(t):
    n = jnp.linalg.norm(t, axis=1, keepdims=True)
    return t / jnp.maximum(n, 1e-12)


def kernel(inputs, labels, W):
    flat = inputs.reshape(-1, _EMB_DIM)
    xn = _l2norm(flat)
    wn = _l2norm(W)
    labels_i32 = labels.astype(jnp.int32)

    sc_gather = _make_sc_gather()
    lab_rows = sc_gather(wn, labels_i32)

    loss2d, idx2d, encodings = _vq_call(xn, wn, lab_rows)

    quantized = sc_gather(W, idx2d.reshape(_N_ROWS))

    return (
        loss2d.reshape(()),
        quantized.reshape(inputs.shape),
        jnp.array(1),
        encodings,
        idx2d,
    )
